# Initial kernel scaffold; baseline (speedup 1.0000x reference)
import jax
import jax.numpy as jnp
from jax.experimental import pallas as pl


def kernel(i_indices, j_indices, neg_indices, wi, wj):
    raise NotImplementedError("write your pallas kernel here")


# speedup vs baseline: 5.0679x; 5.0679x over previous
import jax
import jax.numpy as jnp
from jax import lax
from jax.experimental import pallas as pl
from jax.experimental.pallas import tpu as pltpu
from jax.experimental.pallas import tpu_sc as plsc

D = 64
N_NEG = 20
NDOT = N_NEG + 1
NC, NS = 2, 16
NW = NC * NS
CHUNK = 32


def _sc_body(i_hbm, j_hbm, neg_hbm, wi_hbm, wj_hbm, comb_hbm,
             iidx, jidx, nidx, wirows, wjrows, negrows, accscr, sbuf,
             sem0, sem1):
  B = i_hbm.shape[0]
  per_w = B // NW
  n_chunks = per_w // CHUNK
  nrows_per_chunk = CHUNK * N_NEG // 128
  dots_per_chunk = CHUNK * NDOT
  n_groups = dots_per_chunk // 16

  wid = lax.axis_index("s") * NC + lax.axis_index("c")
  base = wid * per_w

  pltpu.sync_copy(i_hbm.at[pl.ds(base, per_w)], iidx)
  pltpu.sync_copy(j_hbm.at[pl.ds(base, per_w)], jidx)
  nrow0 = wid * (per_w * N_NEG // 128)
  pltpu.sync_copy(neg_hbm.at[pl.ds(nrow0, per_w * N_NEG // 128)], nidx)

  sems = (sem0, sem1)
  lanes = lax.iota(jnp.int32, 16)

  def issue(c, p):
    sem = sems[p]
    descs = [
        pltpu.async_copy(wi_hbm.at[iidx.at[pl.ds(c * CHUNK, CHUNK)]],
                         wirows.at[p], sem),
        pltpu.async_copy(wj_hbm.at[jidx.at[pl.ds(c * CHUNK, CHUNK)]],
                         wjrows.at[p], sem),
    ]
    for r in range(nrows_per_chunk):
      descs.append(
          pltpu.async_copy(wj_hbm.at[nidx.at[c * nrows_per_chunk + r]],
                           negrows.at[p].at[pl.ds(r * 128, 128)], sem))
    return descs

  def compute(c, p):
    def item(b, _):
      wiv = [wirows[p, b, pl.ds(k * 16, 16)] for k in range(4)]
      acc = wiv[0] * wjrows[p, b, pl.ds(0, 16)]
      for k in range(1, 4):
        acc = acc + wiv[k] * wjrows[p, b, pl.ds(k * 16, 16)]
      accscr[pl.ds(b * NDOT * 16, 16)] = acc
      for n in range(N_NEG):
        r = b * N_NEG + n
        nacc = wiv[0] * negrows[p, r, pl.ds(0, 16)]
        for k in range(1, 4):
          nacc = nacc + wiv[k] * negrows[p, r, pl.ds(k * 16, 16)]
        accscr[pl.ds((b * NDOT + 1 + n) * 16, 16)] = nacc
      return 0

    lax.fori_loop(0, CHUNK, item, 0)

    def reduce_group(g, _):
      rows = (g * 16 + lanes) * 16
      red = plsc.load_gather(accscr, [rows])
      for i in range(1, 16):
        red = red + plsc.load_gather(accscr, [rows + i])
      sbuf[pl.ds(c * dots_per_chunk + g * 16, 16)] = red
      return 0

    lax.fori_loop(0, n_groups, reduce_group, 0)

  descs = issue(0, 0)
  for c in range(n_chunks):
    p = c & 1
    nxt = issue(c + 1, 1 - p) if c + 1 < n_chunks else []
    for d in descs:
      d.wait()
    compute(c, p)
    descs = nxt

  pltpu.sync_copy(sbuf, comb_hbm.at[pl.ds(base * NDOT, per_w * NDOT)])


def _sc_scores(i_idx, j_idx, neg2d, wi, wj):
  B = i_idx.shape[0]
  per_w = B // NW
  mesh = plsc.VectorSubcoreMesh(core_axis_name="c", subcore_axis_name="s")
  f = pl.kernel(
      _sc_body,
      out_type=jax.ShapeDtypeStruct((B * NDOT,), jnp.float32),
      mesh=mesh,
      compiler_params=pltpu.CompilerParams(needs_layout_passes=False,
                                           use_tc_tiling_on_sc=False),
      scratch_types=[
          pltpu.VMEM((per_w,), jnp.int32),
          pltpu.VMEM((per_w,), jnp.int32),
          pltpu.VMEM((per_w * N_NEG // 128, 128), jnp.int32),
          pltpu.VMEM((2, CHUNK, D), jnp.float32),
          pltpu.VMEM((2, CHUNK, D), jnp.float32),
          pltpu.VMEM((2, CHUNK * N_NEG, D), jnp.float32),
          pltpu.VMEM((CHUNK * NDOT * 16,), jnp.float32),
          pltpu.VMEM((per_w * NDOT,), jnp.float32),
          pltpu.SemaphoreType.DMA,
          pltpu.SemaphoreType.DMA,
      ],
  )
  return f(i_idx, j_idx, neg2d, wi, wj)


def _tc_loss_body(comb_ref, out_ref):
  rows, cols = comb_ref.shape
  flat = (lax.broadcasted_iota(jnp.int32, (rows, cols), 0) * cols
          + lax.broadcasted_iota(jnp.int32, (rows, cols), 1))
  is_pos = (flat % NDOT) == 0
  s = jnp.clip(comb_ref[...], -10.0, 10.0)
  x = jnp.where(is_pos, -s, s)
  loss = jnp.log1p(jnp.exp(x))
  out_ref[0, 0] = jnp.sum(loss) / (rows * cols // NDOT)


def _tc_loss(comb2d):
  out = pl.pallas_call(
      _tc_loss_body,
      out_shape=jax.ShapeDtypeStruct((1, 1), jnp.float32),
      out_specs=pl.BlockSpec(memory_space=pltpu.SMEM),
  )(comb2d)
  return out[0, 0]


def kernel(i_indices, j_indices, neg_indices, wi, wj):
  B = i_indices.shape[0]
  neg2d = neg_indices.reshape(B * N_NEG // 128, 128)
  comb = _sc_scores(i_indices, j_indices, neg2d, wi, wj)
  return _tc_loss(comb.reshape(B * NDOT // 128, 128))
